# baseline (speedup 1.0000x reference)
import functools

import jax
import jax.numpy as jnp
from jax import lax
from jax.experimental import pallas as pl
from jax.experimental.pallas import tpu as pltpu
from jax.experimental.pallas import tpu_sc as plsc

N_ = 10000
E_ = 320000
F_ = 128
H_ = 128
C_ = 64
L_ = 10
ALPHA_ = 0.9
BETA_ = 1.0 - ALPHA_

NSC_ = 2
NT_ = 16
CH_ = C_ // NSC_
RPT_ = 640
QR_ = 320
NPAD_ = NT_ * RPT_
KG_ = 64
EPT_ = 20480
NG_ = EPT_ // KG_
EPAD_ = NT_ * EPT_


def _mlp_body(x_ref, wi_ref, bi_ref, wo_ref, bo_ref, o_ref):
    h = lax.dot_general(x_ref[...], wi_ref[...], (((1,), (1,)), ((), ())),
                        preferred_element_type=jnp.float32)
    h = jnp.maximum(h + bi_ref[...], 0.0)
    o = lax.dot_general(h, wo_ref[...], (((1,), (1,)), ((), ())),
                        preferred_element_type=jnp.float32)
    o_ref[...] = o + bo_ref[...]


def _mlp(x_pad, W_in, b_in, W_out, b_out):
    grid = NPAD_ // RPT_
    return pl.pallas_call(
        _mlp_body,
        grid=(grid,),
        in_specs=[
            pl.BlockSpec((RPT_, F_), lambda i: (i, 0)),
            pl.BlockSpec((H_, F_), lambda i: (0, 0)),
            pl.BlockSpec((1, H_), lambda i: (0, 0)),
            pl.BlockSpec((C_, H_), lambda i: (0, 0)),
            pl.BlockSpec((1, C_), lambda i: (0, 0)),
        ],
        out_specs=pl.BlockSpec((RPT_, C_), lambda i: (i, 0)),
        out_shape=jax.ShapeDtypeStruct((NPAD_, C_), jnp.float32),
    )(x_pad, W_in, b_in.reshape(1, H_), W_out, b_out.reshape(1, C_))


def _fill(ref, rows, val):
    v16 = jnp.full((16,), val, jnp.float32)

    def fb(i, carry):
        for u in range(4):
            ref[i * 4 + u, pl.ds(0, 16)] = v16
            ref[i * 4 + u, pl.ds(16, 16)] = v16
        return carry

    lax.fori_loop(0, rows // 4, fb, 0)


def _sc_body(x0buf, srcg, dstg, hbuf,
             src_scr, dst_scr, gbuf, astg, xstg, normbuf, acc, hsh,
             gsem, ssem):
    c = lax.axis_index("c")
    s = lax.axis_index("s")
    row0 = c * NPAD_
    base = s * RPT_

    pltpu.sync_copy(srcg.at[s], src_scr)
    pltpu.sync_copy(dstg.at[s], dst_scr)

    _fill(xstg, QR_, 0.0)
    for q in range(RPT_ // QR_):
        lo = base + q * QR_
        pltpu.sync_copy(x0buf.at[pl.ds(row0 + lo, QR_)], astg)
        pltpu.sync_copy(astg, hsh.at[pl.ds(lo, QR_)])
        pltpu.sync_copy(xstg, acc.at[pl.ds(lo, QR_)])
    plsc.subcore_barrier()

    _fill(gbuf.at[0], KG_, 1.0)

    def deg_body(j, carry):
        pltpu.sync_copy(gbuf.at[0], acc.at[dst_scr.at[j]], add=True)
        return carry

    lax.fori_loop(0, NG_, deg_body, 0)
    plsc.subcore_barrier()

    _fill(xstg, QR_, 0.0)
    for q in range(RPT_ // QR_):
        lo = base + q * QR_
        pltpu.sync_copy(acc.at[pl.ds(lo, QR_)], astg)
        pltpu.sync_copy(xstg, acc.at[pl.ds(lo, QR_)])

        def norm_body(v, carry):
            d = astg[v, pl.ds(0, 16)]
            normbuf[q * QR_ + v, pl.ds(0, 16)] = ALPHA_ / jnp.maximum(d, 1.0)
            return carry

        lax.fori_loop(0, QR_, norm_body, 0)
    plsc.subcore_barrier()

    def _g_copy(g, b):
        return pltpu.make_async_copy(
            hsh.at[src_scr.at[pl.ds(g * KG_, KG_)]], gbuf.at[b], gsem.at[b])

    def _s_copy(g, b):
        return pltpu.make_async_copy(
            gbuf.at[b], acc.at[dst_scr.at[g]], ssem.at[b])

    def round_body(r, carry):
        _g_copy(0, 0).start()
        _g_copy(1, 1).start()

        def edge_body(i, carry2):
            g0 = i * 2
            _g_copy(g0, 0).wait()
            _s_copy(g0, 0).start(add=True)
            _g_copy(g0 + 1, 1).wait()
            _s_copy(g0 + 1, 1).start(add=True)
            _s_copy(g0, 0).wait()
            _g_copy(g0 + 2, 0).start()
            _s_copy(g0 + 1, 1).wait()
            _g_copy(g0 + 3, 1).start()
            return carry2

        lax.fori_loop(0, NG_ // 2, edge_body, 0)
        _g_copy(NG_, 0).wait()
        _g_copy(NG_ + 1, 1).wait()
        plsc.subcore_barrier()

        for q in range(RPT_ // QR_):
            lo = base + q * QR_
            pltpu.sync_copy(acc.at[pl.ds(lo, QR_)], astg)
            _fill(xstg, QR_, 0.0)
            pltpu.sync_copy(xstg, acc.at[pl.ds(lo, QR_)])
            pltpu.sync_copy(x0buf.at[pl.ds(row0 + lo, QR_)], xstg)

            def hb(v, carry2):
                nr = normbuf[q * QR_ + v, pl.ds(0, 16)]
                for col in (0, 16):
                    a = astg[v, pl.ds(col, 16)]
                    b = xstg[v, pl.ds(col, 16)]
                    astg[v, pl.ds(col, 16)] = a * nr + b * BETA_
                return carry2

            lax.fori_loop(0, QR_, hb, 0)
            pltpu.sync_copy(astg, hsh.at[pl.ds(lo, QR_)])

            @pl.when(r == L_ - 1)
            def _():
                pltpu.sync_copy(astg, hbuf.at[pl.ds(row0 + lo, QR_)])

        plsc.subcore_barrier()
        return carry

    lax.fori_loop(0, L_, round_body, 0)


def _propagate(x0buf, srcg, dstg):
    mesh = plsc.VectorSubcoreMesh(core_axis_name="c", subcore_axis_name="s")
    return pl.kernel(
        _sc_body,
        out_type=jax.ShapeDtypeStruct((NSC_ * NPAD_, CH_), jnp.float32),
        mesh=mesh,
        compiler_params=pltpu.CompilerParams(use_tc_tiling_on_sc=False),
        scratch_types=[
            pltpu.VMEM((EPT_ + 2 * KG_,), jnp.int32),
            pltpu.VMEM((NG_, KG_), jnp.int32),
            pltpu.VMEM((2, KG_, CH_), jnp.float32),
            pltpu.VMEM((QR_, CH_), jnp.float32),
            pltpu.VMEM((QR_, CH_), jnp.float32),
            pltpu.VMEM((RPT_, 16), jnp.float32),
            pltpu.VMEM_SHARED((NPAD_, CH_), jnp.float32),
            pltpu.VMEM_SHARED((NPAD_, CH_), jnp.float32),
            pltpu.SemaphoreType.DMA((2,)),
            pltpu.SemaphoreType.DMA((2,)),
        ],
    )(x0buf, srcg, dstg)


def kernel(x, edge_index, W_in, b_in, W_out, b_out):
    x_pad = jnp.concatenate(
        [x, jnp.zeros((NPAD_ - N_, F_), jnp.float32)], axis=0)
    x0 = _mlp(x_pad, W_in, b_in, W_out, b_out)
    x0buf = x0.reshape(NPAD_, NSC_, CH_).transpose(1, 0, 2).reshape(
        NSC_ * NPAD_, CH_)

    src = edge_index[0]
    dst = edge_index[1]
    pad = EPAD_ - E_
    src_p = jnp.concatenate([src, jnp.zeros((pad,), jnp.int32)])
    dst_p = jnp.concatenate([dst, jnp.full((pad,), N_, jnp.int32)])
    srcg = jnp.concatenate(
        [src_p.reshape(NT_, EPT_), jnp.zeros((NT_, 2 * KG_), jnp.int32)],
        axis=1)
    dstg = dst_p.reshape(NT_, NG_, KG_)

    hbuf = _propagate(x0buf, srcg, dstg)
    h = hbuf.reshape(NSC_, NPAD_, CH_).transpose(1, 0, 2).reshape(NPAD_, C_)
    return h[:N_]


# speedup vs baseline: 1.2862x; 1.2862x over previous
import functools

import jax
import jax.numpy as jnp
from jax import lax
from jax.experimental import pallas as pl
from jax.experimental.pallas import tpu as pltpu
from jax.experimental.pallas import tpu_sc as plsc

N_ = 10000
E_ = 320000
F_ = 128
H_ = 128
C_ = 64
L_ = 10
ALPHA_ = 0.9
BETA_ = 1.0 - ALPHA_

NSC_ = 2
NT_ = 16
CH_ = C_ // NSC_
RPT_ = 640
QR_ = 320
NPAD_ = NT_ * RPT_
KG_ = 128
EPT_ = 20480
NG_ = EPT_ // KG_
EPAD_ = NT_ * EPT_


def _mlp_body(x_ref, wi_ref, bi_ref, wo_ref, bo_ref, o_ref):
    h = lax.dot_general(x_ref[...], wi_ref[...], (((1,), (1,)), ((), ())),
                        preferred_element_type=jnp.float32)
    h = jnp.maximum(h + bi_ref[...], 0.0)
    o = lax.dot_general(h, wo_ref[...], (((1,), (1,)), ((), ())),
                        preferred_element_type=jnp.float32)
    o_ref[...] = o + bo_ref[...]


def _mlp(x_pad, W_in, b_in, W_out, b_out):
    grid = NPAD_ // RPT_
    return pl.pallas_call(
        _mlp_body,
        grid=(grid,),
        in_specs=[
            pl.BlockSpec((RPT_, F_), lambda i: (i, 0)),
            pl.BlockSpec((H_, F_), lambda i: (0, 0)),
            pl.BlockSpec((1, H_), lambda i: (0, 0)),
            pl.BlockSpec((C_, H_), lambda i: (0, 0)),
            pl.BlockSpec((1, C_), lambda i: (0, 0)),
        ],
        out_specs=pl.BlockSpec((RPT_, C_), lambda i: (i, 0)),
        out_shape=jax.ShapeDtypeStruct((NPAD_, C_), jnp.float32),
    )(x_pad, W_in, b_in.reshape(1, H_), W_out, b_out.reshape(1, C_))


def _fill(ref, rows, val):
    v16 = jnp.full((16,), val, jnp.float32)

    def fb(i, carry):
        for u in range(4):
            ref[i * 4 + u, pl.ds(0, 16)] = v16
            ref[i * 4 + u, pl.ds(16, 16)] = v16
        return carry

    lax.fori_loop(0, rows // 4, fb, 0)


def _sc_body(x0buf, srcg, dstg, hbuf,
             src_scr, dst_scr, gbuf, astg, xstg, normbuf, acc, hsh,
             gsem, ssem):
    c = lax.axis_index("c")
    s = lax.axis_index("s")
    row0 = c * NPAD_
    base = s * RPT_

    pltpu.sync_copy(srcg.at[s], src_scr)
    pltpu.sync_copy(dstg.at[s], dst_scr)

    _fill(xstg, QR_, 0.0)
    for q in range(RPT_ // QR_):
        lo = base + q * QR_
        pltpu.sync_copy(x0buf.at[pl.ds(row0 + lo, QR_)], astg)
        pltpu.sync_copy(astg, hsh.at[pl.ds(lo, QR_)])
        pltpu.sync_copy(xstg, acc.at[pl.ds(lo, QR_)])
    plsc.subcore_barrier()

    _fill(gbuf.at[0], KG_, 1.0)

    def deg_body(j, carry):
        pltpu.sync_copy(gbuf.at[0], acc.at[dst_scr.at[j]], add=True)
        return carry

    lax.fori_loop(0, NG_, deg_body, 0)
    plsc.subcore_barrier()

    _fill(xstg, QR_, 0.0)
    for q in range(RPT_ // QR_):
        lo = base + q * QR_
        pltpu.sync_copy(acc.at[pl.ds(lo, QR_)], astg)
        pltpu.sync_copy(xstg, acc.at[pl.ds(lo, QR_)])

        def norm_body(v, carry):
            d = astg[v, pl.ds(0, 16)]
            normbuf[q * QR_ + v, pl.ds(0, 16)] = ALPHA_ / jnp.maximum(d, 1.0)
            return carry

        lax.fori_loop(0, QR_, norm_body, 0)
    plsc.subcore_barrier()

    def _g_copy(g, b):
        return pltpu.make_async_copy(
            hsh.at[src_scr.at[pl.ds(g * KG_, KG_)]], gbuf.at[b], gsem.at[b])

    def _s_copy(g, b):
        return pltpu.make_async_copy(
            gbuf.at[b], acc.at[dst_scr.at[g]], ssem.at[b])

    def round_body(r, carry):
        for b in range(4):
            _g_copy(b, b).start()

        def edge_body(i, carry2):
            g0 = i * 4
            _g_copy(g0, 0).wait()
            _s_copy(g0, 0).start(add=True)
            _g_copy(g0 + 1, 1).wait()
            _s_copy(g0 + 1, 1).start(add=True)
            _s_copy(g0, 0).wait()
            _g_copy(g0 + 4, 0).start()
            _g_copy(g0 + 2, 2).wait()
            _s_copy(g0 + 2, 2).start(add=True)
            _s_copy(g0 + 1, 1).wait()
            _g_copy(g0 + 5, 1).start()
            _g_copy(g0 + 3, 3).wait()
            _s_copy(g0 + 3, 3).start(add=True)
            _s_copy(g0 + 2, 2).wait()
            _g_copy(g0 + 6, 2).start()
            _s_copy(g0 + 3, 3).wait()
            _g_copy(g0 + 7, 3).start()
            return carry2

        lax.fori_loop(0, NG_ // 4, edge_body, 0)
        for b in range(4):
            _g_copy(NG_ + b, b).wait()
        plsc.subcore_barrier()

        for q in range(RPT_ // QR_):
            lo = base + q * QR_
            pltpu.sync_copy(acc.at[pl.ds(lo, QR_)], astg)
            _fill(xstg, QR_, 0.0)
            pltpu.sync_copy(xstg, acc.at[pl.ds(lo, QR_)])
            pltpu.sync_copy(x0buf.at[pl.ds(row0 + lo, QR_)], xstg)

            def hb(v, carry2):
                nr = normbuf[q * QR_ + v, pl.ds(0, 16)]
                for col in (0, 16):
                    a = astg[v, pl.ds(col, 16)]
                    b = xstg[v, pl.ds(col, 16)]
                    astg[v, pl.ds(col, 16)] = a * nr + b * BETA_
                return carry2

            lax.fori_loop(0, QR_, hb, 0)
            pltpu.sync_copy(astg, hsh.at[pl.ds(lo, QR_)])

            @pl.when(r == L_ - 1)
            def _():
                pltpu.sync_copy(astg, hbuf.at[pl.ds(row0 + lo, QR_)])

        plsc.subcore_barrier()
        return carry

    lax.fori_loop(0, L_, round_body, 0)


def _propagate(x0buf, srcg, dstg):
    mesh = plsc.VectorSubcoreMesh(core_axis_name="c", subcore_axis_name="s")
    return pl.kernel(
        _sc_body,
        out_type=jax.ShapeDtypeStruct((NSC_ * NPAD_, CH_), jnp.float32),
        mesh=mesh,
        compiler_params=pltpu.CompilerParams(use_tc_tiling_on_sc=False),
        scratch_types=[
            pltpu.VMEM((EPT_ + 4 * KG_,), jnp.int32),
            pltpu.VMEM((NG_, KG_), jnp.int32),
            pltpu.VMEM((4, KG_, CH_), jnp.float32),
            pltpu.VMEM((QR_, CH_), jnp.float32),
            pltpu.VMEM((QR_, CH_), jnp.float32),
            pltpu.VMEM((RPT_, 16), jnp.float32),
            pltpu.VMEM_SHARED((NPAD_, CH_), jnp.float32),
            pltpu.VMEM_SHARED((NPAD_, CH_), jnp.float32),
            pltpu.SemaphoreType.DMA((4,)),
            pltpu.SemaphoreType.DMA((4,)),
        ],
    )(x0buf, srcg, dstg)


def kernel(x, edge_index, W_in, b_in, W_out, b_out):
    x_pad = jnp.concatenate(
        [x, jnp.zeros((NPAD_ - N_, F_), jnp.float32)], axis=0)
    x0 = _mlp(x_pad, W_in, b_in, W_out, b_out)
    x0buf = x0.reshape(NPAD_, NSC_, CH_).transpose(1, 0, 2).reshape(
        NSC_ * NPAD_, CH_)

    src = edge_index[0]
    dst = edge_index[1]
    pad = EPAD_ - E_
    src_p = jnp.concatenate([src, jnp.zeros((pad,), jnp.int32)])
    dst_p = jnp.concatenate([dst, jnp.full((pad,), N_, jnp.int32)])
    srcg = jnp.concatenate(
        [src_p.reshape(NT_, EPT_), jnp.zeros((NT_, 4 * KG_), jnp.int32)],
        axis=1)
    dstg = dst_p.reshape(NT_, NG_, KG_)

    hbuf = _propagate(x0buf, srcg, dstg)
    h = hbuf.reshape(NSC_, NPAD_, CH_).transpose(1, 0, 2).reshape(NPAD_, C_)
    return h[:N_]
